# baseline (speedup 1.0000x reference)
import functools

import jax
import jax.numpy as jnp
from jax import lax
from jax.experimental import pallas as pl
from jax.experimental.pallas import tpu as pltpu
from jax.experimental.pallas import tpu_sc as plsc

B = 4096
L = 200
D = 32
NUM_WORKERS = 32
BBLOCK = B // NUM_WORKERS
NBUF = 4
NGROUPS = L // NBUF


def _sc_body(idx_hbm, table_hbm, out_hbm, idx_v, bufs, tb0, tb1, gs0, gs1,
             gs2, gs3, ss0, ss1):
    gsems = (gs0, gs1, gs2, gs3)
    tbufs = (tb0, tb1)
    ssems = (ss0, ss1)
    wid = lax.axis_index("s") * 2 + lax.axis_index("c")
    b0 = pl.multiple_of(wid * BBLOCK, BBLOCK)

    pltpu.sync_copy(idx_hbm.at[:, pl.ds(b0, BBLOCK)], idx_v)

    def fire(l, k):
        pltpu.async_copy(table_hbm.at[idx_v.at[l]], bufs.at[k], gsems[k])

    def wait_gather(k):
        pltpu.make_async_copy(table_hbm.at[pl.ds(0, BBLOCK)], bufs.at[k],
                              gsems[k]).wait()

    def store(l, p):
        pltpu.async_copy(tbufs[p], out_hbm.at[l, :, pl.ds(b0, BBLOCK)],
                         ssems[p])

    def wait_store(p):
        pltpu.make_async_copy(tbufs[p], out_hbm.at[0, :, pl.ds(b0, BBLOCK)],
                              ssems[p]).wait()

    lanes = lax.iota(jnp.int32, 16)
    zmask = (lanes > 0).astype(jnp.float32)
    lanes16 = [lanes + 16 * v for v in range(BBLOCK // 16)]

    def transpose_block(k, p):
        buf = bufs.at[k]
        for d in range(D):
            dvec = lanes * 0 + d
            for v in range(BBLOCK // 16):
                val = plsc.load_gather(buf, [lanes16[v], dvec])
                tbufs[p][d, pl.ds(16 * v, 16)] = val

        @pl.when(wid == 0)
        def _():
            for d in range(D):
                tbufs[p][d, pl.ds(0, 16)] = tbufs[p][d, pl.ds(0, 16)] * zmask

    for k in range(NBUF):
        fire(k, k)

    def group(g, carry):
        l0 = pl.multiple_of(g * NBUF, NBUF)
        for k in range(NBUF):
            l = l0 + k
            p = k % 2
            wait_gather(k)

            @pl.when(g + (1 if k >= 2 else 0) > 0)
            def _():
                wait_store(p)
            transpose_block(k, p)

            @pl.when(l + NBUF < L)
            def _():
                fire(l + NBUF, k)
            store(l, p)
        return carry

    lax.fori_loop(0, NGROUPS, group, 0)
    wait_store(0)
    wait_store(1)


@jax.jit
def _sc_gather(idx_t, table):
    mesh = plsc.VectorSubcoreMesh(core_axis_name="c", subcore_axis_name="s")
    run = functools.partial(
        pl.kernel,
        mesh=mesh,
        out_type=jax.ShapeDtypeStruct((L, D, B), jnp.float32),
        scratch_types=[
            pltpu.VMEM((L, BBLOCK), jnp.int32),
            pltpu.VMEM((NBUF, BBLOCK, D), jnp.float32),
            pltpu.VMEM((D, BBLOCK), jnp.float32),
            pltpu.VMEM((D, BBLOCK), jnp.float32),
            pltpu.SemaphoreType.DMA,
            pltpu.SemaphoreType.DMA,
            pltpu.SemaphoreType.DMA,
            pltpu.SemaphoreType.DMA,
            pltpu.SemaphoreType.DMA,
            pltpu.SemaphoreType.DMA,
        ],
        compiler_params=pltpu.CompilerParams(use_tc_tiling_on_sc=False,
                                             needs_layout_passes=False),
    )(_sc_body)
    return run(idx_t, table)


def kernel(y, table):
    yt = y.astype(jnp.int32).T
    idx_t = jnp.concatenate(
        [jnp.zeros((L, 1), jnp.int32), yt[:, :-1]], axis=1)
    out = _sc_gather(idx_t, table)
    return jnp.transpose(out, (2, 0, 1))


# speedup vs baseline: 1.0008x; 1.0008x over previous
import functools

import jax
import jax.numpy as jnp
from jax import lax
from jax.experimental import pallas as pl
from jax.experimental.pallas import tpu as pltpu
from jax.experimental.pallas import tpu_sc as plsc

B = 4096
L = 200
D = 32
NUM_WORKERS = 32
BBLOCK = B // NUM_WORKERS
NBUF = 4
NGROUPS = L // NBUF


def _sc_body(idx_hbm, table_hbm, out_hbm, idx_v, bufs, tb0, tb1, tb2, tb3,
             gs0, gs1, gs2, gs3, ss0, ss1, ss2, ss3):
    gsems = (gs0, gs1, gs2, gs3)
    tbufs = (tb0, tb1, tb2, tb3)
    ssems = (ss0, ss1, ss2, ss3)
    wid = lax.axis_index("s") * 2 + lax.axis_index("c")
    b0 = pl.multiple_of(wid * BBLOCK, BBLOCK)

    pltpu.sync_copy(idx_hbm.at[:, pl.ds(b0, BBLOCK)], idx_v)

    def fire(l, k):
        pltpu.async_copy(table_hbm.at[idx_v.at[l]], bufs.at[k], gsems[k])

    def wait_gather(k):
        pltpu.make_async_copy(table_hbm.at[pl.ds(0, BBLOCK)], bufs.at[k],
                              gsems[k]).wait()

    def store(l, p):
        pltpu.async_copy(tbufs[p], out_hbm.at[l, :, pl.ds(b0, BBLOCK)],
                         ssems[p])

    def wait_store(p):
        pltpu.make_async_copy(tbufs[p], out_hbm.at[0, :, pl.ds(b0, BBLOCK)],
                              ssems[p]).wait()

    lanes = lax.iota(jnp.int32, 16)
    zmask = (lanes > 0).astype(jnp.float32)
    lanes16 = [lanes + 16 * v for v in range(BBLOCK // 16)]

    def transpose_block(k, p):
        buf = bufs.at[k]
        for d in range(D):
            dvec = lanes * 0 + d
            for v in range(BBLOCK // 16):
                val = plsc.load_gather(buf, [lanes16[v], dvec])
                tbufs[p][d, pl.ds(16 * v, 16)] = val

        @pl.when(wid == 0)
        def _():
            for d in range(D):
                tbufs[p][d, pl.ds(0, 16)] = tbufs[p][d, pl.ds(0, 16)] * zmask

    for k in range(NBUF):
        fire(k, k)

    def group(g, carry):
        l0 = pl.multiple_of(g * NBUF, NBUF)
        for k in range(NBUF):
            l = l0 + k
            p = k
            wait_gather(k)

            @pl.when(g > 0)
            def _():
                wait_store(p)
            transpose_block(k, p)

            @pl.when(l + NBUF < L)
            def _():
                fire(l + NBUF, k)
            store(l, p)
        return carry

    lax.fori_loop(0, NGROUPS, group, 0)
    for p in range(NBUF):
        wait_store(p)


@jax.jit
def _sc_gather(idx_t, table):
    mesh = plsc.VectorSubcoreMesh(core_axis_name="c", subcore_axis_name="s")
    run = functools.partial(
        pl.kernel,
        mesh=mesh,
        out_type=jax.ShapeDtypeStruct((L, D, B), jnp.float32),
        scratch_types=[
            pltpu.VMEM((L, BBLOCK), jnp.int32),
            pltpu.VMEM((NBUF, BBLOCK, D), jnp.float32),
            pltpu.VMEM((D, BBLOCK), jnp.float32),
            pltpu.VMEM((D, BBLOCK), jnp.float32),
            pltpu.VMEM((D, BBLOCK), jnp.float32),
            pltpu.VMEM((D, BBLOCK), jnp.float32),
            pltpu.SemaphoreType.DMA,
            pltpu.SemaphoreType.DMA,
            pltpu.SemaphoreType.DMA,
            pltpu.SemaphoreType.DMA,
            pltpu.SemaphoreType.DMA,
            pltpu.SemaphoreType.DMA,
            pltpu.SemaphoreType.DMA,
            pltpu.SemaphoreType.DMA,
        ],
        compiler_params=pltpu.CompilerParams(use_tc_tiling_on_sc=False,
                                             needs_layout_passes=False),
    )(_sc_body)
    return run(idx_t, table)


def kernel(y, table):
    yt = y.astype(jnp.int32).T
    idx_t = jnp.concatenate(
        [jnp.zeros((L, 1), jnp.int32), yt[:, :-1]], axis=1)
    out = _sc_gather(idx_t, table)
    return jnp.transpose(out, (2, 0, 1))
